# Initial kernel scaffold; baseline (speedup 1.0000x reference)
import jax
import jax.numpy as jnp
from jax.experimental import pallas as pl


def kernel(x, weight):
    raise NotImplementedError("write your pallas kernel here")


# speedup vs baseline: 1.3065x; 1.3065x over previous
import functools

import jax
import jax.numpy as jnp
from jax import lax
from jax.experimental import pallas as pl
from jax.experimental.pallas import tpu as pltpu
from jax.experimental.pallas import tpu_sc as plsc

_G = 128


@functools.cache
def _build(n, d):
    info = plsc.get_sparse_core_info()
    nw = info.num_cores * info.num_subcores
    per_w = n // nw
    n_g = per_w // _G

    mesh = plsc.VectorSubcoreMesh(core_axis_name="c", subcore_axis_name="s")

    @functools.partial(
        pl.kernel,
        out_type=jax.ShapeDtypeStruct((n, d), jnp.float32),
        mesh=mesh,
        compiler_params=pltpu.CompilerParams(use_tc_tiling_on_sc=False),
        scratch_types=[
            pltpu.VMEM((n_g, _G), jnp.int32),
            pltpu.VMEM((_G, d), jnp.float32),
            pltpu.SemaphoreType.DMA,
        ],
    )
    def emb(x_hbm, w_hbm, out_hbm, idx_v, buf, gsem):
        wid = lax.axis_index("s") * info.num_cores + lax.axis_index("c")
        row0 = wid * n_g
        pltpu.sync_copy(x_hbm.at[pl.ds(row0, n_g)], idx_v)

        def chunk(j, carry):
            pltpu.async_copy(w_hbm.at[idx_v.at[j]], buf, gsem).wait()
            pltpu.sync_copy(buf, out_hbm.at[pl.ds((row0 + j) * _G, _G)])
            return carry

        lax.fori_loop(0, n_g, chunk, 0)

    return emb


def kernel(x, weight):
    b, h = x.shape
    _, d = weight.shape
    n = b * h
    x2 = x.reshape(n // _G, _G).astype(jnp.int32)
    out = _build(n, d)(x2, weight)
    return out.reshape(b, h, d)


# speedup vs baseline: 1.5007x; 1.1486x over previous
import functools

import jax
import jax.numpy as jnp
from jax import lax
from jax.experimental import pallas as pl
from jax.experimental.pallas import tpu as pltpu
from jax.experimental.pallas import tpu_sc as plsc

_G = 128
_K = 10


@functools.cache
def _build(n, d):
    info = plsc.get_sparse_core_info()
    nw = info.num_cores * info.num_subcores
    per_w = n // nw
    n_g = per_w // _G
    ch = _K * _G
    n_big = per_w // ch
    n_pair = n_big // 2

    mesh = plsc.VectorSubcoreMesh(core_axis_name="c", subcore_axis_name="s")

    @functools.partial(
        pl.kernel,
        out_type=jax.ShapeDtypeStruct((n, d), jnp.float32),
        mesh=mesh,
        compiler_params=pltpu.CompilerParams(use_tc_tiling_on_sc=False),
        scratch_types=[
            pltpu.VMEM((n_g, _G), jnp.int32),
            pltpu.VMEM((ch, d), jnp.float32),
            pltpu.VMEM((ch, d), jnp.float32),
            pltpu.SemaphoreType.DMA,
            pltpu.SemaphoreType.DMA,
            pltpu.SemaphoreType.DMA,
        ],
    )
    def emb(x_hbm, w_hbm, out_hbm, idx_v, buf0, buf1, gsem0, gsem1, osem):
        wid = lax.axis_index("s") * info.num_cores + lax.axis_index("c")
        row0 = wid * n_g
        out0 = wid * per_w
        pltpu.sync_copy(x_hbm.at[pl.ds(row0, n_g)], idx_v)

        def fire(c, buf, sem):
            for g in range(_K):
                pltpu.async_copy(
                    w_hbm.at[idx_v.at[c * _K + g]],
                    buf.at[pl.ds(g * _G, _G)],
                    sem,
                )

        def drain(buf, sem):
            pltpu.make_async_copy(w_hbm.at[pl.ds(0, ch)], buf, sem).wait()

        def outcopy(c, buf):
            pltpu.async_copy(buf, out_hbm.at[pl.ds(out0 + c * ch, ch)], osem)

        def owait():
            pltpu.make_async_copy(buf0, out_hbm.at[pl.ds(out0, ch)], osem).wait()

        fire(0, buf0, gsem0)
        fire(1, buf1, gsem1)
        drain(buf0, gsem0)
        outcopy(0, buf0)

        def body(u, carry):
            a = 2 * u
            owait()
            fire(a, buf0, gsem0)
            drain(buf1, gsem1)
            outcopy(a - 1, buf1)
            owait()
            fire(a + 1, buf1, gsem1)
            drain(buf0, gsem0)
            outcopy(a, buf0)
            return carry

        lax.fori_loop(1, n_pair, body, 0)

        owait()
        drain(buf1, gsem1)
        outcopy(n_big - 1, buf1)
        owait()

    return emb


def kernel(x, weight):
    b, h = x.shape
    _, d = weight.shape
    n = b * h
    x2 = x.reshape(n // _G, _G).astype(jnp.int32)
    out = _build(n, d)(x2, weight)
    return out.reshape(b, h, d)
